# baseline (device time: 769574 ns/iter reference)
import jax
from jax import lax
from jax.experimental import pallas as pl
from jax.experimental.pallas import tpu as pltpu

NZ = 4


def kernel(x):
    m, n = x.shape
    blk = n // NZ
    half = m // 2

    def body(x_ref, out_ref, relay_ref, copy_sem, send_sems, recv_sems):
        my_x = lax.axis_index("x")
        my_y = lax.axis_index("y")
        my_z = lax.axis_index("z")
        up = (my_z + 1) % NZ
        dn = (my_z - 1) % NZ
        opp = (my_z + 2) % NZ

        barrier = pltpu.get_barrier_semaphore()
        for nbr in (up, dn):
            pl.semaphore_signal(
                barrier, inc=1,
                device_id=(my_x, my_y, nbr),
                device_id_type=pl.DeviceIdType.MESH,
            )
        pl.semaphore_wait(barrier, 2)

        local = pltpu.make_async_copy(
            x_ref.at[:, pl.ds(my_z * blk, blk)],
            out_ref.at[pl.ds(my_z * m, m), :],
            copy_sem,
        )
        local.start()

        r1_up = pltpu.make_async_remote_copy(
            src_ref=x_ref.at[pl.ds(0, half), pl.ds(opp * blk, blk)],
            dst_ref=relay_ref.at[0],
            send_sem=send_sems.at[2],
            recv_sem=recv_sems.at[2],
            device_id=(my_x, my_y, up),
            device_id_type=pl.DeviceIdType.MESH,
        )
        r1_up.start()
        r1_dn = pltpu.make_async_remote_copy(
            src_ref=x_ref.at[pl.ds(half, half), pl.ds(opp * blk, blk)],
            dst_ref=relay_ref.at[1],
            send_sem=send_sems.at[3],
            recv_sem=recv_sems.at[3],
            device_id=(my_x, my_y, dn),
            device_id_type=pl.DeviceIdType.MESH,
        )
        r1_dn.start()

        d_up = pltpu.make_async_remote_copy(
            src_ref=x_ref.at[:, pl.ds(up * blk, blk)],
            dst_ref=out_ref.at[pl.ds(my_z * m, m), :],
            send_sem=send_sems.at[0],
            recv_sem=recv_sems.at[0],
            device_id=(my_x, my_y, up),
            device_id_type=pl.DeviceIdType.MESH,
        )
        d_up.start()
        d_dn = pltpu.make_async_remote_copy(
            src_ref=x_ref.at[:, pl.ds(dn * blk, blk)],
            dst_ref=out_ref.at[pl.ds(my_z * m, m), :],
            send_sem=send_sems.at[1],
            recv_sem=recv_sems.at[1],
            device_id=(my_x, my_y, dn),
            device_id_type=pl.DeviceIdType.MESH,
        )
        d_dn.start()

        r1_up.wait_recv()
        f_up = pltpu.make_async_remote_copy(
            src_ref=relay_ref.at[0],
            dst_ref=out_ref.at[pl.ds(dn * m, half), :],
            send_sem=send_sems.at[4],
            recv_sem=recv_sems.at[4],
            device_id=(my_x, my_y, up),
            device_id_type=pl.DeviceIdType.MESH,
        )
        f_up.start()

        r1_dn.wait_recv()
        f_dn = pltpu.make_async_remote_copy(
            src_ref=relay_ref.at[1],
            dst_ref=out_ref.at[pl.ds(up * m + half, half), :],
            send_sem=send_sems.at[5],
            recv_sem=recv_sems.at[5],
            device_id=(my_x, my_y, dn),
            device_id_type=pl.DeviceIdType.MESH,
        )
        f_dn.start()

        d_up.wait_recv()
        d_dn.wait_recv()

        fin_top = pltpu.make_async_remote_copy(
            src_ref=relay_ref.at[0],
            dst_ref=out_ref.at[pl.ds(opp * m, half), :],
            send_sem=send_sems.at[4],
            recv_sem=recv_sems.at[4],
            device_id=(my_x, my_y, dn),
            device_id_type=pl.DeviceIdType.MESH,
        )
        fin_top.wait_recv()
        fin_bot = pltpu.make_async_remote_copy(
            src_ref=relay_ref.at[1],
            dst_ref=out_ref.at[pl.ds(opp * m + half, half), :],
            send_sem=send_sems.at[5],
            recv_sem=recv_sems.at[5],
            device_id=(my_x, my_y, up),
            device_id_type=pl.DeviceIdType.MESH,
        )
        fin_bot.wait_recv()

        local.wait()
        r1_up.wait_send()
        r1_dn.wait_send()
        d_up.wait_send()
        d_dn.wait_send()
        f_up.wait_send()
        f_dn.wait_send()

    return pl.pallas_call(
        body,
        out_shape=jax.ShapeDtypeStruct((NZ * m, blk), x.dtype),
        in_specs=[pl.BlockSpec(memory_space=pl.ANY)],
        out_specs=pl.BlockSpec(memory_space=pl.ANY),
        scratch_shapes=[
            pltpu.VMEM((2, half, blk), x.dtype),
            pltpu.SemaphoreType.DMA,
            pltpu.SemaphoreType.DMA((6,)),
            pltpu.SemaphoreType.DMA((6,)),
        ],
        compiler_params=pltpu.CompilerParams(collective_id=0),
    )(x)


# device time: 590073 ns/iter; 1.3042x vs baseline; 1.3042x over previous
import jax
from jax import lax
from jax.experimental import pallas as pl
from jax.experimental.pallas import tpu as pltpu

NZ = 4


def kernel(x):
    m, n = x.shape
    blk = n // NZ

    def body(x_ref, out_ref, copy_sem, send_sems, recv_sems):
        my_x = lax.axis_index("x")
        my_y = lax.axis_index("y")
        my_z = lax.axis_index("z")

        barrier = pltpu.get_barrier_semaphore()
        for d in range(1, NZ):
            peer = (my_z + d) % NZ
            pl.semaphore_signal(
                barrier, inc=1,
                device_id=(my_x, my_y, peer),
                device_id_type=pl.DeviceIdType.MESH,
            )
        pl.semaphore_wait(barrier, NZ - 1)

        local = pltpu.make_async_copy(
            x_ref.at[:, pl.ds(my_z * blk, blk)],
            out_ref.at[pl.ds(my_z * m, m), :],
            copy_sem,
        )
        local.start()

        sends = []
        for d in range(1, NZ):
            peer = (my_z + d) % NZ
            rdma = pltpu.make_async_remote_copy(
                src_ref=x_ref.at[:, pl.ds(peer * blk, blk)],
                dst_ref=out_ref.at[pl.ds(my_z * m, m), :],
                send_sem=send_sems.at[d - 1],
                recv_sem=recv_sems.at[d - 1],
                device_id=(my_x, my_y, peer),
                device_id_type=pl.DeviceIdType.MESH,
            )
            rdma.start()
            sends.append(rdma)

        local.wait()
        for rdma in sends:
            rdma.wait_send()

        for d in range(1, NZ):
            src_z = (my_z - d) % NZ
            recv = pltpu.make_async_remote_copy(
                src_ref=x_ref.at[:, pl.ds(my_z * blk, blk)],
                dst_ref=out_ref.at[pl.ds(src_z * m, m), :],
                send_sem=send_sems.at[d - 1],
                recv_sem=recv_sems.at[d - 1],
                device_id=(my_x, my_y, src_z),
                device_id_type=pl.DeviceIdType.MESH,
            )
            recv.wait_recv()

    return pl.pallas_call(
        body,
        out_shape=jax.ShapeDtypeStruct((NZ * m, blk), x.dtype),
        in_specs=[pl.BlockSpec(memory_space=pl.ANY)],
        out_specs=pl.BlockSpec(memory_space=pl.ANY),
        scratch_shapes=[
            pltpu.SemaphoreType.DMA,
            pltpu.SemaphoreType.DMA((NZ - 1,)),
            pltpu.SemaphoreType.DMA((NZ - 1,)),
        ],
        compiler_params=pltpu.CompilerParams(collective_id=0),
    )(x)
